# baseline (speedup 1.0000x reference)
import functools

import jax
import jax.numpy as jnp
from jax import lax
from jax.experimental import pallas as pl
from jax.experimental.pallas import tpu as pltpu
from jax.experimental.pallas import tpu_sc as plsc

V = 1024
D = 64
BATCH = 16
SEQ = 4096
B = BATCH * SEQ
NC, NS = 2, 16
NW = NC * NS
IDS_PER_ROW = 128
ROWS_TOTAL = B // IDS_PER_ROW
ROWS_PER_W = ROWS_TOTAL // NW
ROWS_PER_CHUNK = 2
IDS_PER_CHUNK = ROWS_PER_CHUNK * IDS_PER_ROW
CHUNKS = ROWS_PER_W // ROWS_PER_CHUNK
IDS_PER_W = ROWS_PER_W * IDS_PER_ROW
NBUF = 2


def _tables_body(inv_ref, cos_ref, sin_ref):
    f = inv_ref[...]
    cos_ref[...] = jnp.cos(f)
    sin_ref[...] = jnp.sin(f)


def _make_tables(inv_freq):
    return pl.pallas_call(
        _tables_body,
        out_shape=(
            jax.ShapeDtypeStruct((V, D), jnp.float32),
            jax.ShapeDtypeStruct((V, D), jnp.float32),
        ),
    )(inv_freq)


def _gather_body(cos_tab, sin_tab, idx_hbm, cos_out, sin_out,
                 idx_v, cos_buf, sin_buf, gsem0, gsem1, wsem0, wsem1):
    wid = lax.axis_index("s") * NC + lax.axis_index("c")
    bi = wid // 2
    off0 = (wid % 2) * IDS_PER_W
    gsems = (gsem0, gsem1)
    wsems = (wsem0, wsem1)

    def outer(g, carry):
        for b in range(NBUF):
            c = g * NBUF + b
            row0 = wid * ROWS_PER_W + c * ROWS_PER_CHUNK
            off = off0 + c * IDS_PER_CHUNK
            dst_c = cos_out.at[bi, pl.ds(off, IDS_PER_CHUNK)]
            dst_s = sin_out.at[bi, pl.ds(off, IDS_PER_CHUNK)]

            @pl.when(g >= 1)
            def _():
                pltpu.make_async_copy(cos_buf.at[b], dst_c, wsems[b]).wait()
                pltpu.make_async_copy(sin_buf.at[b], dst_s, wsems[b]).wait()

            pltpu.sync_copy(idx_hbm.at[pl.ds(row0, ROWS_PER_CHUNK)],
                            idx_v.at[b])
            cps = []
            for j in range(ROWS_PER_CHUNK):
                dst = pl.ds(j * IDS_PER_ROW, IDS_PER_ROW)
                cps.append(pltpu.async_copy(
                    cos_tab.at[idx_v.at[b, j]], cos_buf.at[b, dst], gsems[b]))
                cps.append(pltpu.async_copy(
                    sin_tab.at[idx_v.at[b, j]], sin_buf.at[b, dst], gsems[b]))
            for cp in cps:
                cp.wait()
            pltpu.async_copy(cos_buf.at[b], dst_c, wsems[b])
            pltpu.async_copy(sin_buf.at[b], dst_s, wsems[b])
        return carry

    lax.fori_loop(0, CHUNKS // NBUF, outer, 0)

    for b in range(NBUF):
        c = CHUNKS - NBUF + b
        off = off0 + c * IDS_PER_CHUNK
        dst_c = cos_out.at[bi, pl.ds(off, IDS_PER_CHUNK)]
        dst_s = sin_out.at[bi, pl.ds(off, IDS_PER_CHUNK)]
        pltpu.make_async_copy(cos_buf.at[b], dst_c, wsems[b]).wait()
        pltpu.make_async_copy(sin_buf.at[b], dst_s, wsems[b]).wait()


@functools.cache
def _make_gather():
    return pl.kernel(
        _gather_body,
        out_type=(
            jax.ShapeDtypeStruct((BATCH, SEQ, D), jnp.float32),
            jax.ShapeDtypeStruct((BATCH, SEQ, D), jnp.float32),
        ),
        mesh=plsc.VectorSubcoreMesh(core_axis_name="c", subcore_axis_name="s"),
        compiler_params=pltpu.CompilerParams(use_tc_tiling_on_sc=False),
        scratch_types=[
            pltpu.VMEM((NBUF, ROWS_PER_CHUNK, IDS_PER_ROW), jnp.int32),
            pltpu.VMEM((NBUF, IDS_PER_CHUNK, D), jnp.float32),
            pltpu.VMEM((NBUF, IDS_PER_CHUNK, D), jnp.float32),
            pltpu.SemaphoreType.DMA,
            pltpu.SemaphoreType.DMA,
            pltpu.SemaphoreType.DMA,
            pltpu.SemaphoreType.DMA,
        ],
    )


def kernel(x, position_ids, inv_freq):
    cos_tab, sin_tab = _make_tables(inv_freq.astype(jnp.float32))
    idx = position_ids.reshape(ROWS_TOTAL, IDS_PER_ROW).astype(jnp.int32)
    cos_f, sin_f = _make_gather()(cos_tab, sin_tab, idx)
    return (cos_f.astype(x.dtype), sin_f.astype(x.dtype))


# speedup vs baseline: 1.0284x; 1.0284x over previous
import functools

import jax
import jax.numpy as jnp
from jax import lax
from jax.experimental import pallas as pl
from jax.experimental.pallas import tpu as pltpu
from jax.experimental.pallas import tpu_sc as plsc

V = 1024
D = 64
D2 = 2 * D
BATCH = 16
SEQ = 4096
B = BATCH * SEQ
NC, NS = 2, 16
NW = NC * NS
IDS_PER_ROW = 128
ROWS_TOTAL = B // IDS_PER_ROW
ROWS_PER_W = ROWS_TOTAL // NW
ROWS_PER_CHUNK = 2
IDS_PER_CHUNK = ROWS_PER_CHUNK * IDS_PER_ROW
CHUNKS = ROWS_PER_W // ROWS_PER_CHUNK
IDS_PER_W = ROWS_PER_W * IDS_PER_ROW
NBUF = 2
SPLIT_BLK = 2048


def _tables_body(inv_ref, tab_ref):
    f = inv_ref[...]
    tab_ref[:, :D] = jnp.cos(f)
    tab_ref[:, D:] = jnp.sin(f)


def _make_tables(inv_freq):
    return pl.pallas_call(
        _tables_body,
        out_shape=jax.ShapeDtypeStruct((V, D2), jnp.float32),
    )(inv_freq)


def _gather_body(tab, idx_hbm, comb_out, idx_v, buf, gsem0, gsem1,
                 wsem0, wsem1):
    wid = lax.axis_index("s") * NC + lax.axis_index("c")
    base = wid * IDS_PER_W
    gsems = (gsem0, gsem1)
    wsems = (wsem0, wsem1)

    def outer(g, carry):
        for b in range(NBUF):
            c = g * NBUF + b
            row0 = wid * ROWS_PER_W + c * ROWS_PER_CHUNK
            off = base + c * IDS_PER_CHUNK
            dst = comb_out.at[pl.ds(off, IDS_PER_CHUNK)]

            @pl.when(g >= 1)
            def _():
                pltpu.make_async_copy(buf.at[b], dst, wsems[b]).wait()

            pltpu.sync_copy(idx_hbm.at[pl.ds(row0, ROWS_PER_CHUNK)],
                            idx_v.at[b])
            cps = []
            for j in range(ROWS_PER_CHUNK):
                d = pl.ds(j * IDS_PER_ROW, IDS_PER_ROW)
                cps.append(pltpu.async_copy(
                    tab.at[idx_v.at[b, j]], buf.at[b, d], gsems[b]))
            for cp in cps:
                cp.wait()
            pltpu.async_copy(buf.at[b], dst, wsems[b])
        return carry

    lax.fori_loop(0, CHUNKS // NBUF, outer, 0)

    for b in range(NBUF):
        c = CHUNKS - NBUF + b
        off = base + c * IDS_PER_CHUNK
        dst = comb_out.at[pl.ds(off, IDS_PER_CHUNK)]
        pltpu.make_async_copy(buf.at[b], dst, wsems[b]).wait()


@functools.cache
def _make_gather():
    return pl.kernel(
        _gather_body,
        out_type=jax.ShapeDtypeStruct((B, D2), jnp.float32),
        mesh=plsc.VectorSubcoreMesh(core_axis_name="c", subcore_axis_name="s"),
        compiler_params=pltpu.CompilerParams(use_tc_tiling_on_sc=False),
        scratch_types=[
            pltpu.VMEM((NBUF, ROWS_PER_CHUNK, IDS_PER_ROW), jnp.int32),
            pltpu.VMEM((NBUF, IDS_PER_CHUNK, D2), jnp.float32),
            pltpu.SemaphoreType.DMA,
            pltpu.SemaphoreType.DMA,
            pltpu.SemaphoreType.DMA,
            pltpu.SemaphoreType.DMA,
        ],
    )


def _split_body(comb_ref, cos_ref, sin_ref):
    rows = comb_ref[...]
    cos_ref[...] = rows[:, :D]
    sin_ref[...] = rows[:, D:]


def _split(comb):
    return pl.pallas_call(
        _split_body,
        grid=(B // SPLIT_BLK,),
        in_specs=[pl.BlockSpec((SPLIT_BLK, D2), lambda i: (i, 0))],
        out_specs=(pl.BlockSpec((SPLIT_BLK, D), lambda i: (i, 0)),
                   pl.BlockSpec((SPLIT_BLK, D), lambda i: (i, 0))),
        out_shape=(jax.ShapeDtypeStruct((B, D), jnp.float32),
                   jax.ShapeDtypeStruct((B, D), jnp.float32)),
    )(comb)


def kernel(x, position_ids, inv_freq):
    tab = _make_tables(inv_freq.astype(jnp.float32))
    idx = position_ids.reshape(ROWS_TOTAL, IDS_PER_ROW).astype(jnp.int32)
    comb = _make_gather()(tab, idx)
    cos_f, sin_f = _split(comb)
    shape = position_ids.shape + (D,)
    return (cos_f.reshape(shape).astype(x.dtype),
            sin_f.reshape(shape).astype(x.dtype))
